# baseline (speedup 1.0000x reference)
import jax
import jax.numpy as jnp
from jax import lax
from jax.experimental import pallas as pl
from jax.experimental.pallas import tpu as pltpu
from jax.experimental.pallas import tpu_sc as plsc

_NUM_CORES = 2
_NUM_SUBCORES = 16
_NUM_WORKERS = _NUM_CORES * _NUM_SUBCORES
_CHUNK = 64


def _make_body(batch, seq, width, rows_per_worker, n_chunks):
    def body(pe_hbm, out_hbm, buf):
        wid = lax.axis_index("s") * _NUM_CORES + lax.axis_index("c")
        base = wid * rows_per_worker
        for j in range(n_chunks):
            r0 = base + j * _CHUNK
            pltpu.sync_copy(pe_hbm.at[pl.ds(r0, _CHUNK)], buf)
            for b in range(batch):
                pltpu.sync_copy(buf, out_hbm.at[pl.ds(b * seq + r0, _CHUNK)])
    return body


def kernel(inputs, position_embeddings):
    batch, seq, width = inputs.shape
    pe = position_embeddings[:seq, :]
    rows_per_worker = seq // _NUM_WORKERS
    n_chunks = rows_per_worker // _CHUNK
    run = pl.kernel(
        _make_body(batch, seq, width, rows_per_worker, n_chunks),
        out_type=jax.ShapeDtypeStruct((batch * seq, width), jnp.float32),
        mesh=plsc.VectorSubcoreMesh(core_axis_name="c", subcore_axis_name="s"),
        scratch_types=[pltpu.VMEM((_CHUNK, width), jnp.float32)],
    )
    out = run(pe)
    return out.reshape(batch, seq, width)
